# baseline (speedup 1.0000x reference)
import functools

import jax
import jax.numpy as jnp
from jax.experimental import pallas as pl
from jax.experimental.pallas import tpu as pltpu

_ALPHA = 0.05
_BETA = 3.0
_K = 2
_EPS = 1e-8


def _encoder_body(x_ref, af_ref, ab_ref, e1_ref, e2_ref, wg1_ref, wg2_ref,
                  wzr_ref, bzr_ref, wc_ref, bc_ref, out_ref, *, P, N, H):
    f32 = jnp.float32

    Af = jnp.maximum(af_ref[...], 0.0)
    Afn = Af / (jnp.sum(Af, axis=-1, keepdims=True) + _EPS)
    Ab = jnp.maximum(ab_ref[...], 0.0)
    Abn = Ab / (jnp.sum(Ab, axis=-1, keepdims=True) + _EPS)
    Afnb = Afn.astype(jnp.bfloat16)
    Abnb = Abn.astype(jnp.bfloat16)

    e1 = e1_ref[...]
    e2 = e2_ref[...]
    wg1 = wg1_ref[...]
    wg2 = wg2_ref[...]
    wzr = wzr_ref[...]
    bzr_v = bzr_ref[...]
    wc = wc_ref[...]
    bc_v = bc_ref[...]

    bf16 = jnp.bfloat16

    def gconv(y, sup, W, bvec):
        outs = [y]
        for A in sup:
            hh = y
            for _ in range(_K):
                hh = _ALPHA * y + (1.0 - _ALPHA) * jnp.dot(
                    A, hh.astype(bf16), preferred_element_type=f32)
                outs.append(hh)
        ho = jnp.concatenate(outs, axis=-1)
        return jnp.dot(ho, W, preferred_element_type=f32) + bvec

    def step(t, h):
        xt = x_ref[0, t]
        inp = jnp.concatenate([xt, h], axis=-1)
        f1 = jnp.tanh(jnp.dot(inp, wg1, preferred_element_type=f32) * e1)
        f2 = jnp.tanh(jnp.dot(inp, wg2, preferred_element_type=f32) * e2)
        f1b = f1.astype(bf16)
        f2b = f2.astype(bf16)
        M1 = jax.lax.dot_general(f1b, f2b, (((1,), (1,)), ((), ())),
                                 preferred_element_type=f32)
        M2 = jax.lax.dot_general(f2b, f1b, (((1,), (1,)), ((), ())),
                                 preferred_element_type=f32)
        Tm = jnp.tanh(_BETA * (M1 - M2))
        Ar = jnp.maximum(Tm, 0.0)
        Ac = jnp.maximum(-Tm, 0.0)
        Ad1 = Ar / (jnp.sum(Ar, axis=-1, keepdims=True) + _EPS)
        Ad2 = Ac / (jnp.sum(Ac, axis=-1, keepdims=True) + _EPS)
        sup = (Afnb, Abnb, Ad1.astype(bf16), Ad2.astype(bf16))
        zr = jax.nn.sigmoid(gconv(inp, sup, wzr, bzr_v))
        z = zr[:, :H]
        r = zr[:, H:]
        cin = jnp.concatenate([xt, r * h], axis=-1)
        c = jnp.tanh(gconv(cin, sup, wc, bc_v))
        return z * h + (1.0 - z) * c

    h0 = jnp.zeros((N, H), dtype=f32)
    hf = jax.lax.fori_loop(0, P, step, h0)
    out_ref[0] = hf


def kernel(x, A_fwd, A_bwd, E1, E2, Wg1, Wg2, Wzr, bzr, Wc, bc):
    B, P, N, C = x.shape
    H = Wc.shape[1]
    EMB = E1.shape[1]
    D = C + H
    feat = Wzr.shape[0]

    bzr2 = bzr.reshape(1, -1)
    bc2 = bc.reshape(1, -1)

    body = functools.partial(_encoder_body, P=P, N=N, H=H)
    grid = (B,)
    out = pl.pallas_call(
        body,
        grid=grid,
        in_specs=[
            pl.BlockSpec((1, P, N, C), lambda b: (b, 0, 0, 0)),
            pl.BlockSpec((N, N), lambda b: (0, 0)),
            pl.BlockSpec((N, N), lambda b: (0, 0)),
            pl.BlockSpec((N, EMB), lambda b: (0, 0)),
            pl.BlockSpec((N, EMB), lambda b: (0, 0)),
            pl.BlockSpec((D, EMB), lambda b: (0, 0)),
            pl.BlockSpec((D, EMB), lambda b: (0, 0)),
            pl.BlockSpec((feat, 2 * H), lambda b: (0, 0)),
            pl.BlockSpec((1, 2 * H), lambda b: (0, 0)),
            pl.BlockSpec((feat, H), lambda b: (0, 0)),
            pl.BlockSpec((1, H), lambda b: (0, 0)),
        ],
        out_specs=pl.BlockSpec((1, N, H), lambda b: (b, 0, 0)),
        out_shape=jax.ShapeDtypeStruct((B, N, H), x.dtype),
        compiler_params=pltpu.CompilerParams(
            dimension_semantics=("parallel",),
        ),
    )(x, A_fwd, A_bwd, E1, E2, Wg1, Wg2, Wzr, bzr2, Wc, bc2)
    return out


# speedup vs baseline: 1.1869x; 1.1869x over previous
import functools

import jax
import jax.numpy as jnp
from jax.experimental import pallas as pl
from jax.experimental.pallas import tpu as pltpu

_ALPHA = 0.05
_BETA = 3.0
_K = 2
_EPS = 1e-8


def _encoder_body(x_ref, af_ref, ab_ref, e1_ref, e2_ref, wg1_ref, wg2_ref,
                  wzr_ref, bzr_ref, wc_ref, bc_ref, out_ref, *, P, N, H):
    f32 = jnp.float32
    bf16 = jnp.bfloat16

    Af = jnp.maximum(af_ref[...], 0.0)
    Afnb = (Af / (jnp.sum(Af, axis=-1, keepdims=True) + _EPS)).astype(bf16)
    Ab = jnp.maximum(ab_ref[...], 0.0)
    Abnb = (Ab / (jnp.sum(Ab, axis=-1, keepdims=True) + _EPS)).astype(bf16)

    e1 = e1_ref[...]
    e2 = e2_ref[...]
    wg1 = wg1_ref[...]
    wg2 = wg2_ref[...]
    wzr = wzr_ref[...]
    bzr_v = bzr_ref[...]
    wc = wc_ref[...]
    bc_v = bc_ref[...]
    ones_col = jnp.ones((N, 1), dtype=bf16)

    def step(t, h):
        xt = x_ref[0, t]
        inp = jnp.concatenate([xt, h], axis=-1)
        f1 = jnp.tanh(jnp.dot(inp, wg1, preferred_element_type=f32) * e1)
        f2 = jnp.tanh(jnp.dot(inp, wg2, preferred_element_type=f32) * e2)
        f1b = (_BETA * f1).astype(bf16)
        f2b = f2.astype(bf16)
        M1 = jax.lax.dot_general(f1b, f2b, (((1,), (1,)), ((), ())),
                                 preferred_element_type=f32)
        M2 = jax.lax.dot_general(f2b, f1b, (((1,), (1,)), ((), ())),
                                 preferred_element_type=f32)
        Tb = jnp.tanh(M1 - M2).astype(bf16)
        Arb = jnp.maximum(Tb, 0)
        Acmb = jnp.minimum(Tb, 0)

        inp_b = inp.astype(bf16)
        y1 = jnp.concatenate([inp_b, ones_col], axis=-1)
        V = jnp.dot(Arb, y1, preferred_element_type=f32)
        U = jnp.dot(Acmb, y1, preferred_element_type=f32)
        rs = 1.0 / (V[:, -1:] + _EPS)
        cs = 1.0 / (U[:, -1:] - _EPS)

        def prop_af(yb):
            return jnp.dot(Afnb, yb, preferred_element_type=f32)

        def prop_ab(yb):
            return jnp.dot(Abnb, yb, preferred_element_type=f32)

        def prop_d1(yb):
            return rs * jnp.dot(Arb, yb, preferred_element_type=f32)

        def prop_d2(yb):
            return cs * jnp.dot(Acmb, yb, preferred_element_type=f32)

        props = (prop_af, prop_ab, prop_d1, prop_d2)

        def gconv(y, p1_list, W, bvec):
            outs = [y]
            for p1, prop in zip(p1_list, props):
                h1 = _ALPHA * y + (1.0 - _ALPHA) * p1
                h2 = _ALPHA * y + (1.0 - _ALPHA) * prop(h1.astype(bf16))
                outs.append(h1)
                outs.append(h2)
            ho = jnp.concatenate(outs, axis=-1)
            return jnp.dot(ho, W, preferred_element_type=f32) + bvec

        p1_zr = (prop_af(inp_b), prop_ab(inp_b), rs * V[:, :-1], cs * U[:, :-1])
        zr = jax.nn.sigmoid(gconv(inp, p1_zr, wzr, bzr_v))
        z = zr[:, :H]
        r = zr[:, H:]
        cin = jnp.concatenate([xt, r * h], axis=-1)
        cin_b = cin.astype(bf16)
        p1_c = tuple(prop(cin_b) for prop in props)
        c = jnp.tanh(gconv(cin, p1_c, wc, bc_v))
        return z * h + (1.0 - z) * c

    h0 = jnp.zeros((N, H), dtype=f32)
    hf = jax.lax.fori_loop(0, P, step, h0)
    out_ref[0] = hf


def kernel(x, A_fwd, A_bwd, E1, E2, Wg1, Wg2, Wzr, bzr, Wc, bc):
    B, P, N, C = x.shape
    H = Wc.shape[1]
    EMB = E1.shape[1]
    D = C + H
    feat = Wzr.shape[0]

    bzr2 = bzr.reshape(1, -1)
    bc2 = bc.reshape(1, -1)

    body = functools.partial(_encoder_body, P=P, N=N, H=H)
    grid = (B,)
    out = pl.pallas_call(
        body,
        grid=grid,
        in_specs=[
            pl.BlockSpec((1, P, N, C), lambda b: (b, 0, 0, 0)),
            pl.BlockSpec((N, N), lambda b: (0, 0)),
            pl.BlockSpec((N, N), lambda b: (0, 0)),
            pl.BlockSpec((N, EMB), lambda b: (0, 0)),
            pl.BlockSpec((N, EMB), lambda b: (0, 0)),
            pl.BlockSpec((D, EMB), lambda b: (0, 0)),
            pl.BlockSpec((D, EMB), lambda b: (0, 0)),
            pl.BlockSpec((feat, 2 * H), lambda b: (0, 0)),
            pl.BlockSpec((1, 2 * H), lambda b: (0, 0)),
            pl.BlockSpec((feat, H), lambda b: (0, 0)),
            pl.BlockSpec((1, H), lambda b: (0, 0)),
        ],
        out_specs=pl.BlockSpec((1, N, H), lambda b: (b, 0, 0)),
        out_shape=jax.ShapeDtypeStruct((B, N, H), x.dtype),
        compiler_params=pltpu.CompilerParams(
            dimension_semantics=("parallel",),
        ),
    )(x, A_fwd, A_bwd, E1, E2, Wg1, Wg2, Wzr, bzr2, Wc, bc2)
    return out


# speedup vs baseline: 1.4076x; 1.1860x over previous
import functools

import jax
import jax.numpy as jnp
from jax.experimental import pallas as pl
from jax.experimental.pallas import tpu as pltpu

_ALPHA = 0.05
_BETA = 3.0
_K = 2
_EPS = 1e-8


def _encoder_body(x_ref, af_ref, ab_ref, e1_ref, e2_ref, wg1_ref, wg2_ref,
                  wzr_ref, bzr_ref, wc_ref, bc_ref, out_ref, *, B, P, N, C, H):
    f32 = jnp.float32
    bf16 = jnp.bfloat16

    Afb = jnp.maximum(af_ref[...], 0)
    Abb = jnp.maximum(ab_ref[...], 0)
    sf = 1.0 / (jnp.sum(Afb.astype(f32), axis=-1, keepdims=True) + _EPS)
    sb = 1.0 / (jnp.sum(Abb.astype(f32), axis=-1, keepdims=True) + _EPS)

    e1 = e1_ref[...]
    e2 = e2_ref[...]
    wg1 = wg1_ref[...]
    wg2 = wg2_ref[...]
    wzr = wzr_ref[...]
    bzr_v = bzr_ref[...]
    wc = wc_ref[...]
    bc_v = bc_ref[...]
    ones_col = jnp.ones((N, 1), dtype=bf16)
    D = wg1.shape[0]

    def apply_static(A16, scale, y_list):
        Yb = jnp.concatenate(y_list, axis=-1).astype(bf16)
        Pm = scale * jnp.dot(A16, Yb, preferred_element_type=f32)
        return [Pm[:, b * D:(b + 1) * D] for b in range(B)]

    def k2(y, p1):
        return _ALPHA * y + (1.0 - _ALPHA) * p1

    def step(t, hs):
        xts = [x_ref[t, :, b * C:(b + 1) * C] for b in range(B)]
        inps = [jnp.concatenate([xts[b], hs[b]], axis=-1) for b in range(B)]

        pf1 = apply_static(Afb, sf, inps)
        pb1 = apply_static(Abb, sb, inps)
        h1f = [k2(inps[b], pf1[b]) for b in range(B)]
        h1b = [k2(inps[b], pb1[b]) for b in range(B)]
        pf2 = apply_static(Afb, sf, h1f)
        pb2 = apply_static(Abb, sb, h1b)
        h2f = [k2(inps[b], pf2[b]) for b in range(B)]
        h2b = [k2(inps[b], pb2[b]) for b in range(B)]

        zs, cins = [], []
        cd1, cd2 = [], []
        for b in range(B):
            inp = inps[b]
            f1 = jnp.tanh(jnp.dot(inp, wg1, preferred_element_type=f32) * e1)
            f2 = jnp.tanh(jnp.dot(inp, wg2, preferred_element_type=f32) * e2)
            f1b = (_BETA * f1).astype(bf16)
            f2b = f2.astype(bf16)
            M1 = jax.lax.dot_general(f1b, f2b, (((1,), (1,)), ((), ())),
                                     preferred_element_type=f32)
            M2 = jax.lax.dot_general(f2b, f1b, (((1,), (1,)), ((), ())),
                                     preferred_element_type=f32)
            Tb = jnp.tanh(M1 - M2).astype(bf16)
            Arb = jnp.maximum(Tb, 0)
            Acmb = jnp.minimum(Tb, 0)
            y1 = jnp.concatenate([inp.astype(bf16), ones_col], axis=-1)
            V = jnp.dot(Arb, y1, preferred_element_type=f32)
            U = jnp.dot(Acmb, y1, preferred_element_type=f32)
            rs = 1.0 / (V[:, -1:] + _EPS)
            cs = 1.0 / (U[:, -1:] - _EPS)

            h1d1 = k2(inp, rs * V[:, :-1])
            h2d1 = k2(inp, rs * jnp.dot(Arb, h1d1.astype(bf16),
                                        preferred_element_type=f32))
            h1d2 = k2(inp, cs * U[:, :-1])
            h2d2 = k2(inp, cs * jnp.dot(Acmb, h1d2.astype(bf16),
                                        preferred_element_type=f32))
            ho = jnp.concatenate(
                [inp, h1f[b], h2f[b], h1b[b], h2b[b],
                 h1d1, h2d1, h1d2, h2d2], axis=-1)
            zr = jax.nn.sigmoid(jnp.dot(ho, wzr, preferred_element_type=f32)
                                + bzr_v)
            z = zr[:, :H]
            r = zr[:, H:]
            zs.append(z)
            cin = jnp.concatenate([xts[b], r * hs[b]], axis=-1)
            cins.append(cin)
            cb = cin.astype(bf16)
            c1d1 = k2(cin, rs * jnp.dot(Arb, cb, preferred_element_type=f32))
            c2d1 = k2(cin, rs * jnp.dot(Arb, c1d1.astype(bf16),
                                        preferred_element_type=f32))
            c1d2 = k2(cin, cs * jnp.dot(Acmb, cb, preferred_element_type=f32))
            c2d2 = k2(cin, cs * jnp.dot(Acmb, c1d2.astype(bf16),
                                        preferred_element_type=f32))
            cd1.append((c1d1, c2d1))
            cd2.append((c1d2, c2d2))

        qf1 = apply_static(Afb, sf, cins)
        qb1 = apply_static(Abb, sb, cins)
        g1f = [k2(cins[b], qf1[b]) for b in range(B)]
        g1b = [k2(cins[b], qb1[b]) for b in range(B)]
        qf2 = apply_static(Afb, sf, g1f)
        qb2 = apply_static(Abb, sb, g1b)
        g2f = [k2(cins[b], qf2[b]) for b in range(B)]
        g2b = [k2(cins[b], qb2[b]) for b in range(B)]

        new_hs = []
        for b in range(B):
            ho = jnp.concatenate(
                [cins[b], g1f[b], g2f[b], g1b[b], g2b[b],
                 cd1[b][0], cd1[b][1], cd2[b][0], cd2[b][1]], axis=-1)
            c = jnp.tanh(jnp.dot(ho, wc, preferred_element_type=f32) + bc_v)
            new_hs.append(zs[b] * hs[b] + (1.0 - zs[b]) * c)
        return tuple(new_hs)

    h0 = tuple(jnp.zeros((N, H), dtype=f32) for _ in range(B))
    hf = jax.lax.fori_loop(0, P, step, h0)
    for b in range(B):
        out_ref[b] = hf[b]


def kernel(x, A_fwd, A_bwd, E1, E2, Wg1, Wg2, Wzr, bzr, Wc, bc):
    B, P, N, C = x.shape
    H = Wc.shape[1]
    EMB = E1.shape[1]
    D = C + H
    feat = Wzr.shape[0]

    xr = x.transpose(1, 2, 0, 3).reshape(P, N, B * C)
    af16 = A_fwd.astype(jnp.bfloat16)
    ab16 = A_bwd.astype(jnp.bfloat16)
    bzr2 = bzr.reshape(1, -1)
    bc2 = bc.reshape(1, -1)

    body = functools.partial(_encoder_body, B=B, P=P, N=N, C=C, H=H)
    out = pl.pallas_call(
        body,
        grid=(1,),
        in_specs=[
            pl.BlockSpec((P, N, B * C), lambda i: (0, 0, 0)),
            pl.BlockSpec((N, N), lambda i: (0, 0)),
            pl.BlockSpec((N, N), lambda i: (0, 0)),
            pl.BlockSpec((N, EMB), lambda i: (0, 0)),
            pl.BlockSpec((N, EMB), lambda i: (0, 0)),
            pl.BlockSpec((D, EMB), lambda i: (0, 0)),
            pl.BlockSpec((D, EMB), lambda i: (0, 0)),
            pl.BlockSpec((feat, 2 * H), lambda i: (0, 0)),
            pl.BlockSpec((1, 2 * H), lambda i: (0, 0)),
            pl.BlockSpec((feat, H), lambda i: (0, 0)),
            pl.BlockSpec((1, H), lambda i: (0, 0)),
        ],
        out_specs=pl.BlockSpec((B, N, H), lambda i: (0, 0, 0)),
        out_shape=jax.ShapeDtypeStruct((B, N, H), x.dtype),
    )(xr, af16, ab16, E1, E2, Wg1, Wg2, Wzr, bzr2, Wc, bc2)
    return out


# speedup vs baseline: 1.4739x; 1.0471x over previous
import functools

import jax
import jax.numpy as jnp
from jax.experimental import pallas as pl
from jax.experimental.pallas import tpu as pltpu

_ALPHA = 0.05
_BETA = 3.0
_K = 2
_EPS = 1e-8


def _encoder_body(x_ref, af_ref, ab_ref, e1_ref, e2_ref, wg1_ref, wg2_ref,
                  wzr_ref, bzr_ref, wc_ref, bc_ref, out_ref, *, B, P, N, C, H):
    f32 = jnp.float32
    bf16 = jnp.bfloat16

    Afb = jnp.maximum(af_ref[...], 0)
    Abb = jnp.maximum(ab_ref[...], 0)
    sf = 1.0 / (jnp.sum(Afb.astype(f32), axis=-1, keepdims=True) + _EPS)
    sb = 1.0 / (jnp.sum(Abb.astype(f32), axis=-1, keepdims=True) + _EPS)

    e1 = e1_ref[...]
    e2 = e2_ref[...]
    wg1 = wg1_ref[...]
    wg2 = wg2_ref[...]
    wzr = wzr_ref[...].astype(jnp.bfloat16)
    bzr_v = bzr_ref[...]
    wc = wc_ref[...].astype(jnp.bfloat16)
    bc_v = bc_ref[...]
    ones_col = jnp.ones((N, 1), dtype=bf16)
    D = wg1.shape[0]

    def apply_static(A16, scale, y_list):
        Yb = jnp.concatenate(y_list, axis=-1).astype(bf16)
        Pm = scale * jnp.dot(A16, Yb, preferred_element_type=f32)
        return [Pm[:, b * D:(b + 1) * D] for b in range(B)]

    def k2(y, p1):
        return _ALPHA * y + (1.0 - _ALPHA) * p1

    def step(t, hs):
        xts = [x_ref[t, :, b * C:(b + 1) * C] for b in range(B)]
        inps = [jnp.concatenate([xts[b], hs[b]], axis=-1) for b in range(B)]

        pf1 = apply_static(Afb, sf, inps)
        pb1 = apply_static(Abb, sb, inps)
        h1f = [k2(inps[b], pf1[b]) for b in range(B)]
        h1b = [k2(inps[b], pb1[b]) for b in range(B)]
        pf2 = apply_static(Afb, sf, h1f)
        pb2 = apply_static(Abb, sb, h1b)
        h2f = [k2(inps[b], pf2[b]) for b in range(B)]
        h2b = [k2(inps[b], pb2[b]) for b in range(B)]

        zs, cins = [], []
        cd1, cd2 = [], []
        for b in range(B):
            inp = inps[b]
            f1 = jnp.tanh(jnp.dot(inp, wg1, preferred_element_type=f32) * e1)
            f2 = jnp.tanh(jnp.dot(inp, wg2, preferred_element_type=f32) * e2)
            f1b = (_BETA * f1).astype(bf16)
            f2b = f2.astype(bf16)
            G1 = jnp.concatenate([f1b, -f2b], axis=-1)
            G2 = jnp.concatenate([f2b, f1b], axis=-1)
            Ma = jax.lax.dot_general(G1, G2, (((1,), (1,)), ((), ())),
                                     preferred_element_type=f32)
            Tb = jnp.tanh(Ma).astype(bf16)
            Arb = jnp.maximum(Tb, 0)
            Acmb = jnp.minimum(Tb, 0)
            y1 = jnp.concatenate([inp.astype(bf16), ones_col], axis=-1)
            V = jnp.dot(Arb, y1, preferred_element_type=f32)
            U = jnp.dot(Acmb, y1, preferred_element_type=f32)
            rs = 1.0 / (V[:, -1:] + _EPS)
            cs = 1.0 / (U[:, -1:] - _EPS)

            h1d1 = k2(inp, rs * V[:, :-1])
            h2d1 = k2(inp, rs * jnp.dot(Arb, h1d1.astype(bf16),
                                        preferred_element_type=f32))
            h1d2 = k2(inp, cs * U[:, :-1])
            h2d2 = k2(inp, cs * jnp.dot(Acmb, h1d2.astype(bf16),
                                        preferred_element_type=f32))
            ho = jnp.concatenate(
                [inp, h1f[b], h2f[b], h1b[b], h2b[b],
                 h1d1, h2d1, h1d2, h2d2], axis=-1).astype(bf16)
            zr = jax.nn.sigmoid(jnp.dot(ho, wzr, preferred_element_type=f32)
                                + bzr_v)
            z = zr[:, :H]
            r = zr[:, H:]
            zs.append(z)
            cin = jnp.concatenate([xts[b], r * hs[b]], axis=-1)
            cins.append(cin)
            cb = cin.astype(bf16)
            c1d1 = k2(cin, rs * jnp.dot(Arb, cb, preferred_element_type=f32))
            c2d1 = k2(cin, rs * jnp.dot(Arb, c1d1.astype(bf16),
                                        preferred_element_type=f32))
            c1d2 = k2(cin, cs * jnp.dot(Acmb, cb, preferred_element_type=f32))
            c2d2 = k2(cin, cs * jnp.dot(Acmb, c1d2.astype(bf16),
                                        preferred_element_type=f32))
            cd1.append((c1d1, c2d1))
            cd2.append((c1d2, c2d2))

        qf1 = apply_static(Afb, sf, cins)
        qb1 = apply_static(Abb, sb, cins)
        g1f = [k2(cins[b], qf1[b]) for b in range(B)]
        g1b = [k2(cins[b], qb1[b]) for b in range(B)]
        qf2 = apply_static(Afb, sf, g1f)
        qb2 = apply_static(Abb, sb, g1b)
        g2f = [k2(cins[b], qf2[b]) for b in range(B)]
        g2b = [k2(cins[b], qb2[b]) for b in range(B)]

        new_hs = []
        for b in range(B):
            ho = jnp.concatenate(
                [cins[b], g1f[b], g2f[b], g1b[b], g2b[b],
                 cd1[b][0], cd1[b][1], cd2[b][0], cd2[b][1]],
                axis=-1).astype(bf16)
            c = jnp.tanh(jnp.dot(ho, wc, preferred_element_type=f32) + bc_v)
            new_hs.append(zs[b] * hs[b] + (1.0 - zs[b]) * c)
        return tuple(new_hs)

    h0 = tuple(jnp.zeros((N, H), dtype=f32) for _ in range(B))
    hf = jax.lax.fori_loop(0, P, step, h0)
    for b in range(B):
        out_ref[b] = hf[b]


def kernel(x, A_fwd, A_bwd, E1, E2, Wg1, Wg2, Wzr, bzr, Wc, bc):
    B, P, N, C = x.shape
    H = Wc.shape[1]
    EMB = E1.shape[1]
    D = C + H
    feat = Wzr.shape[0]

    xr = x.transpose(1, 2, 0, 3).reshape(P, N, B * C)
    af16 = A_fwd.astype(jnp.bfloat16)
    ab16 = A_bwd.astype(jnp.bfloat16)
    bzr2 = bzr.reshape(1, -1)
    bc2 = bc.reshape(1, -1)

    body = functools.partial(_encoder_body, B=B, P=P, N=N, C=C, H=H)
    out = pl.pallas_call(
        body,
        grid=(1,),
        in_specs=[
            pl.BlockSpec((P, N, B * C), lambda i: (0, 0, 0)),
            pl.BlockSpec((N, N), lambda i: (0, 0)),
            pl.BlockSpec((N, N), lambda i: (0, 0)),
            pl.BlockSpec((N, EMB), lambda i: (0, 0)),
            pl.BlockSpec((N, EMB), lambda i: (0, 0)),
            pl.BlockSpec((D, EMB), lambda i: (0, 0)),
            pl.BlockSpec((D, EMB), lambda i: (0, 0)),
            pl.BlockSpec((feat, 2 * H), lambda i: (0, 0)),
            pl.BlockSpec((1, 2 * H), lambda i: (0, 0)),
            pl.BlockSpec((feat, H), lambda i: (0, 0)),
            pl.BlockSpec((1, H), lambda i: (0, 0)),
        ],
        out_specs=pl.BlockSpec((B, N, H), lambda i: (0, 0, 0)),
        out_shape=jax.ShapeDtypeStruct((B, N, H), x.dtype),
    )(xr, af16, ab16, E1, E2, Wg1, Wg2, Wzr, bzr2, Wc, bc2)
    return out


# speedup vs baseline: 1.5112x; 1.0253x over previous
import functools

import jax
import jax.numpy as jnp
from jax.experimental import pallas as pl
from jax.experimental.pallas import tpu as pltpu

_ALPHA = 0.05
_BETA = 3.0
_K = 2
_EPS = 1e-8


def _encoder_body(x_ref, af_ref, ab_ref, e1_ref, e2_ref, wg1_ref, wg2_ref,
                  wzr_ref, bzr_ref, wc_ref, bc_ref, out_ref, *, B, P, N, C, H):
    f32 = jnp.float32
    bf16 = jnp.bfloat16

    Afb = jnp.maximum(af_ref[...], 0)
    Abb = jnp.maximum(ab_ref[...], 0)
    sf = 1.0 / (jnp.sum(Afb.astype(f32), axis=-1, keepdims=True) + _EPS)
    sb = 1.0 / (jnp.sum(Abb.astype(f32), axis=-1, keepdims=True) + _EPS)

    e1 = e1_ref[...]
    e2 = e2_ref[...]
    wg1 = wg1_ref[...]
    wg2 = wg2_ref[...]
    wzr = wzr_ref[...].astype(jnp.bfloat16)
    bzr_v = bzr_ref[...]
    wc = wc_ref[...].astype(jnp.bfloat16)
    bc_v = bc_ref[...]
    ones_col = jnp.ones((N, 1), dtype=bf16)
    D = wg1.shape[0]

    def static_chain(A16, scale, Ym):
        P1 = scale * jnp.dot(A16, Ym.astype(bf16), preferred_element_type=f32)
        H1 = _ALPHA * Ym + (1.0 - _ALPHA) * P1
        P2 = scale * jnp.dot(A16, H1.astype(bf16), preferred_element_type=f32)
        H2 = _ALPHA * Ym + (1.0 - _ALPHA) * P2
        return H1, H2

    def k2(y, p1):
        return _ALPHA * y + (1.0 - _ALPHA) * p1

    def blk(Am, b):
        return Am[:, b * D:(b + 1) * D]

    def step(t, hs):
        xts = [x_ref[t, :, b * C:(b + 1) * C] for b in range(B)]
        inps = [jnp.concatenate([xts[b], hs[b]], axis=-1) for b in range(B)]
        Y0 = jnp.concatenate(inps, axis=-1)

        H1f, H2f = static_chain(Afb, sf, Y0)
        H1b, H2b = static_chain(Abb, sb, Y0)

        zs, cins = [], []
        cd1, cd2 = [], []
        for b in range(B):
            inp = inps[b]
            f1 = jnp.tanh(jnp.dot(inp, wg1, preferred_element_type=f32) * e1)
            f2 = jnp.tanh(jnp.dot(inp, wg2, preferred_element_type=f32) * e2)
            f1b = (_BETA * f1).astype(bf16)
            f2b = f2.astype(bf16)
            G1 = jnp.concatenate([f1b, -f2b], axis=-1)
            G2 = jnp.concatenate([f2b, f1b], axis=-1)
            Ma = jax.lax.dot_general(G1, G2, (((1,), (1,)), ((), ())),
                                     preferred_element_type=f32)
            Tb = jnp.tanh(Ma).astype(bf16)
            Arb = jnp.maximum(Tb, 0)
            Acmb = jnp.minimum(Tb, 0)
            y1 = jnp.concatenate([inp.astype(bf16), ones_col], axis=-1)
            V = jnp.dot(Arb, y1, preferred_element_type=f32)
            U = jnp.dot(Acmb, y1, preferred_element_type=f32)
            rs = 1.0 / (V[:, -1:] + _EPS)
            cs = 1.0 / (U[:, -1:] - _EPS)

            h1d1 = k2(inp, rs * V[:, :-1])
            h2d1 = k2(inp, rs * jnp.dot(Arb, h1d1.astype(bf16),
                                        preferred_element_type=f32))
            h1d2 = k2(inp, cs * U[:, :-1])
            h2d2 = k2(inp, cs * jnp.dot(Acmb, h1d2.astype(bf16),
                                        preferred_element_type=f32))
            ho = jnp.concatenate(
                [inp, blk(H1f, b), blk(H2f, b), blk(H1b, b), blk(H2b, b),
                 h1d1, h2d1, h1d2, h2d2], axis=-1).astype(bf16)
            zr = jax.nn.sigmoid(jnp.dot(ho, wzr, preferred_element_type=f32)
                                + bzr_v)
            z = zr[:, :H]
            r = zr[:, H:]
            zs.append(z)
            cin = jnp.concatenate([xts[b], r * hs[b]], axis=-1)
            cins.append(cin)
            cb = cin.astype(bf16)
            c1d1 = k2(cin, rs * jnp.dot(Arb, cb, preferred_element_type=f32))
            c2d1 = k2(cin, rs * jnp.dot(Arb, c1d1.astype(bf16),
                                        preferred_element_type=f32))
            c1d2 = k2(cin, cs * jnp.dot(Acmb, cb, preferred_element_type=f32))
            c2d2 = k2(cin, cs * jnp.dot(Acmb, c1d2.astype(bf16),
                                        preferred_element_type=f32))
            cd1.append((c1d1, c2d1))
            cd2.append((c1d2, c2d2))

        Yc = jnp.concatenate(cins, axis=-1)
        G1f, G2f = static_chain(Afb, sf, Yc)
        G1b, G2b = static_chain(Abb, sb, Yc)

        new_hs = []
        for b in range(B):
            ho = jnp.concatenate(
                [cins[b], blk(G1f, b), blk(G2f, b), blk(G1b, b), blk(G2b, b),
                 cd1[b][0], cd1[b][1], cd2[b][0], cd2[b][1]],
                axis=-1).astype(bf16)
            c = jnp.tanh(jnp.dot(ho, wc, preferred_element_type=f32) + bc_v)
            new_hs.append(zs[b] * hs[b] + (1.0 - zs[b]) * c)
        return tuple(new_hs)

    h0 = tuple(jnp.zeros((N, H), dtype=f32) for _ in range(B))
    hf = jax.lax.fori_loop(0, P, step, h0)
    for b in range(B):
        out_ref[b] = hf[b]


def kernel(x, A_fwd, A_bwd, E1, E2, Wg1, Wg2, Wzr, bzr, Wc, bc):
    B, P, N, C = x.shape
    H = Wc.shape[1]
    EMB = E1.shape[1]
    D = C + H
    feat = Wzr.shape[0]

    xr = x.transpose(1, 2, 0, 3).reshape(P, N, B * C)
    af16 = A_fwd.astype(jnp.bfloat16)
    ab16 = A_bwd.astype(jnp.bfloat16)
    bzr2 = bzr.reshape(1, -1)
    bc2 = bc.reshape(1, -1)

    body = functools.partial(_encoder_body, B=B, P=P, N=N, C=C, H=H)
    out = pl.pallas_call(
        body,
        grid=(1,),
        in_specs=[
            pl.BlockSpec((P, N, B * C), lambda i: (0, 0, 0)),
            pl.BlockSpec((N, N), lambda i: (0, 0)),
            pl.BlockSpec((N, N), lambda i: (0, 0)),
            pl.BlockSpec((N, EMB), lambda i: (0, 0)),
            pl.BlockSpec((N, EMB), lambda i: (0, 0)),
            pl.BlockSpec((D, EMB), lambda i: (0, 0)),
            pl.BlockSpec((D, EMB), lambda i: (0, 0)),
            pl.BlockSpec((feat, 2 * H), lambda i: (0, 0)),
            pl.BlockSpec((1, 2 * H), lambda i: (0, 0)),
            pl.BlockSpec((feat, H), lambda i: (0, 0)),
            pl.BlockSpec((1, H), lambda i: (0, 0)),
        ],
        out_specs=pl.BlockSpec((B, N, H), lambda i: (0, 0, 0)),
        out_shape=jax.ShapeDtypeStruct((B, N, H), x.dtype),
    )(xr, af16, ab16, E1, E2, Wg1, Wg2, Wzr, bzr2, Wc, bc2)
    return out


# speedup vs baseline: 1.5118x; 1.0004x over previous
import functools

import jax
import jax.numpy as jnp
from jax.experimental import pallas as pl
from jax.experimental.pallas import tpu as pltpu

_ALPHA = 0.05
_BETA = 3.0
_K = 2
_EPS = 1e-8


def _encoder_body(x_ref, af_ref, ab_ref, e1_ref, e2_ref, wg1_ref, wg2_ref,
                  wzr_ref, bzr_ref, wc_ref, bc_ref, out_ref, *, B, P, N, C, H):
    f32 = jnp.float32
    bf16 = jnp.bfloat16

    Afb = jnp.maximum(af_ref[...], 0)
    Abb = jnp.maximum(ab_ref[...], 0)
    sf = 1.0 / (jnp.sum(Afb.astype(f32), axis=-1, keepdims=True) + _EPS)
    sb = 1.0 / (jnp.sum(Abb.astype(f32), axis=-1, keepdims=True) + _EPS)

    e1 = e1_ref[...]
    e2 = e2_ref[...]
    wg1 = wg1_ref[...].astype(jnp.bfloat16)
    wg2 = wg2_ref[...].astype(jnp.bfloat16)
    wzr = wzr_ref[...].astype(jnp.bfloat16)
    bzr_v = bzr_ref[...]
    wc = wc_ref[...].astype(jnp.bfloat16)
    bc_v = bc_ref[...]
    ones_col = jnp.ones((N, 1), dtype=bf16)
    D = wg1.shape[0]

    def static_chain(A16, scale, Ym):
        P1 = scale * jnp.dot(A16, Ym.astype(bf16), preferred_element_type=f32)
        H1 = _ALPHA * Ym + (1.0 - _ALPHA) * P1
        P2 = scale * jnp.dot(A16, H1.astype(bf16), preferred_element_type=f32)
        H2 = _ALPHA * Ym + (1.0 - _ALPHA) * P2
        return H1, H2

    def k2(y, p1):
        return _ALPHA * y + (1.0 - _ALPHA) * p1

    def blk(Am, b):
        return Am[:, b * D:(b + 1) * D]

    def step(t, hs):
        xts = [x_ref[t, :, b * C:(b + 1) * C] for b in range(B)]
        inps = [jnp.concatenate([xts[b], hs[b]], axis=-1) for b in range(B)]
        Y0 = jnp.concatenate(inps, axis=-1)

        H1f, H2f = static_chain(Afb, sf, Y0)
        H1b, H2b = static_chain(Abb, sb, Y0)

        zs, cins = [], []
        cd1, cd2 = [], []
        for b in range(B):
            inp = inps[b]
            inpb = inp.astype(bf16)
            f1 = jnp.tanh(jnp.dot(inpb, wg1, preferred_element_type=f32) * e1)
            f2 = jnp.tanh(jnp.dot(inpb, wg2, preferred_element_type=f32) * e2)
            f1b = (_BETA * f1).astype(bf16)
            f2b = f2.astype(bf16)
            G1 = jnp.concatenate([f1b, -f2b], axis=-1)
            G2 = jnp.concatenate([f2b, f1b], axis=-1)
            Ma = jax.lax.dot_general(G1, G2, (((1,), (1,)), ((), ())),
                                     preferred_element_type=f32)
            Tb = jnp.tanh(Ma).astype(bf16)
            Arb = jnp.maximum(Tb, 0)
            Acmb = jnp.minimum(Tb, 0)
            y1 = jnp.concatenate([inpb, ones_col], axis=-1)
            V = jnp.dot(Arb, y1, preferred_element_type=f32)
            U = jnp.dot(Acmb, y1, preferred_element_type=f32)
            rs = 1.0 / (V[:, -1:] + _EPS)
            cs = 1.0 / (U[:, -1:] - _EPS)

            h1d1 = k2(inp, rs * V[:, :-1])
            h2d1 = k2(inp, rs * jnp.dot(Arb, h1d1.astype(bf16),
                                        preferred_element_type=f32))
            h1d2 = k2(inp, cs * U[:, :-1])
            h2d2 = k2(inp, cs * jnp.dot(Acmb, h1d2.astype(bf16),
                                        preferred_element_type=f32))
            ho = jnp.concatenate(
                [inp, blk(H1f, b), blk(H2f, b), blk(H1b, b), blk(H2b, b),
                 h1d1, h2d1, h1d2, h2d2], axis=-1).astype(bf16)
            zr = jax.nn.sigmoid(jnp.dot(ho, wzr, preferred_element_type=f32)
                                + bzr_v)
            z = zr[:, :H]
            r = zr[:, H:]
            zs.append(z)
            cin = jnp.concatenate([xts[b], r * hs[b]], axis=-1)
            cins.append(cin)
            cb = cin.astype(bf16)
            c1d1 = k2(cin, rs * jnp.dot(Arb, cb, preferred_element_type=f32))
            c2d1 = k2(cin, rs * jnp.dot(Arb, c1d1.astype(bf16),
                                        preferred_element_type=f32))
            c1d2 = k2(cin, cs * jnp.dot(Acmb, cb, preferred_element_type=f32))
            c2d2 = k2(cin, cs * jnp.dot(Acmb, c1d2.astype(bf16),
                                        preferred_element_type=f32))
            cd1.append((c1d1, c2d1))
            cd2.append((c1d2, c2d2))

        Yc = jnp.concatenate(cins, axis=-1)
        G1f, G2f = static_chain(Afb, sf, Yc)
        G1b, G2b = static_chain(Abb, sb, Yc)

        new_hs = []
        for b in range(B):
            ho = jnp.concatenate(
                [cins[b], blk(G1f, b), blk(G2f, b), blk(G1b, b), blk(G2b, b),
                 cd1[b][0], cd1[b][1], cd2[b][0], cd2[b][1]],
                axis=-1).astype(bf16)
            c = jnp.tanh(jnp.dot(ho, wc, preferred_element_type=f32) + bc_v)
            new_hs.append(zs[b] * hs[b] + (1.0 - zs[b]) * c)
        return tuple(new_hs)

    h0 = tuple(jnp.zeros((N, H), dtype=f32) for _ in range(B))
    hf = jax.lax.fori_loop(0, P, step, h0)
    for b in range(B):
        out_ref[b] = hf[b]


def kernel(x, A_fwd, A_bwd, E1, E2, Wg1, Wg2, Wzr, bzr, Wc, bc):
    B, P, N, C = x.shape
    H = Wc.shape[1]
    EMB = E1.shape[1]
    D = C + H
    feat = Wzr.shape[0]

    xr = x.transpose(1, 2, 0, 3).reshape(P, N, B * C)
    af16 = A_fwd.astype(jnp.bfloat16)
    ab16 = A_bwd.astype(jnp.bfloat16)
    bzr2 = bzr.reshape(1, -1)
    bc2 = bc.reshape(1, -1)

    body = functools.partial(_encoder_body, B=B, P=P, N=N, C=C, H=H)
    out = pl.pallas_call(
        body,
        grid=(1,),
        in_specs=[
            pl.BlockSpec((P, N, B * C), lambda i: (0, 0, 0)),
            pl.BlockSpec((N, N), lambda i: (0, 0)),
            pl.BlockSpec((N, N), lambda i: (0, 0)),
            pl.BlockSpec((N, EMB), lambda i: (0, 0)),
            pl.BlockSpec((N, EMB), lambda i: (0, 0)),
            pl.BlockSpec((D, EMB), lambda i: (0, 0)),
            pl.BlockSpec((D, EMB), lambda i: (0, 0)),
            pl.BlockSpec((feat, 2 * H), lambda i: (0, 0)),
            pl.BlockSpec((1, 2 * H), lambda i: (0, 0)),
            pl.BlockSpec((feat, H), lambda i: (0, 0)),
            pl.BlockSpec((1, H), lambda i: (0, 0)),
        ],
        out_specs=pl.BlockSpec((B, N, H), lambda i: (0, 0, 0)),
        out_shape=jax.ShapeDtypeStruct((B, N, H), x.dtype),
    )(xr, af16, ab16, E1, E2, Wg1, Wg2, Wzr, bzr2, Wc, bc2)
    return out


# speedup vs baseline: 1.6617x; 1.0991x over previous
import functools

import jax
import jax.numpy as jnp
from jax.experimental import pallas as pl
from jax.experimental.pallas import tpu as pltpu

_ALPHA = 0.05
_BETA = 3.0
_K = 2
_EPS = 1e-8


def _encoder_body(x_ref, af_ref, ab_ref, e1_ref, e2_ref, wg1_ref, wg2_ref,
                  wzr_ref, bzr_ref, wc_ref, bc_ref, out_ref, *, B, P, N, C, H):
    f32 = jnp.float32
    bf16 = jnp.bfloat16

    Afb = jnp.maximum(af_ref[...], 0)
    Abb = jnp.maximum(ab_ref[...], 0)
    sf = 1.0 / (jnp.sum(Afb.astype(f32), axis=-1, keepdims=True) + _EPS)
    sb = 1.0 / (jnp.sum(Abb.astype(f32), axis=-1, keepdims=True) + _EPS)

    e1 = e1_ref[...]
    e2 = e2_ref[...]
    wg1 = wg1_ref[...].astype(jnp.bfloat16)
    wg2 = wg2_ref[...].astype(jnp.bfloat16)
    wzr = wzr_ref[...].astype(jnp.bfloat16)
    bzr_v = bzr_ref[...]
    wc = wc_ref[...].astype(jnp.bfloat16)
    bc_v = bc_ref[...]
    ones_col = jnp.ones((N, 1), dtype=bf16)
    D = wg1.shape[0]

    def static_chain(A16, scale, Ym):
        P1 = scale * jnp.dot(A16, Ym.astype(bf16), preferred_element_type=f32)
        H1 = _ALPHA * Ym + (1.0 - _ALPHA) * P1
        P2 = scale * jnp.dot(A16, H1.astype(bf16), preferred_element_type=f32)
        H2 = _ALPHA * Ym + (1.0 - _ALPHA) * P2
        return H1, H2

    def k2(y, p1):
        return _ALPHA * y + (1.0 - _ALPHA) * p1

    def blk(Am, b):
        return Am[:, b * D:(b + 1) * D]

    def step(t, hs):
        xts = [x_ref[t, :, b * C:(b + 1) * C] for b in range(B)]
        inps = [jnp.concatenate([xts[b], hs[b]], axis=-1) for b in range(B)]
        Y0 = jnp.concatenate(inps, axis=-1)

        H1f, H2f = static_chain(Afb, sf, Y0)
        H1b, H2b = static_chain(Abb, sb, Y0)

        zs, cins = [None] * B, [None] * B
        cd1, cd2 = [None] * B, [None] * B
        for g in range(0, B, 2):
            grp = range(g, min(g + 2, B))
            Ar_d, Acm_d, rs_d, cs_d, h1d1_d, h1d2_d = {}, {}, {}, {}, {}, {}
            for b in grp:
                inp = inps[b]
                inpb = inp.astype(bf16)
                f1 = jnp.tanh(jnp.dot(inpb, wg1, preferred_element_type=f32)
                              * e1)
                f2 = jnp.tanh(jnp.dot(inpb, wg2, preferred_element_type=f32)
                              * e2)
                f1b = (_BETA * f1).astype(bf16)
                f2b = f2.astype(bf16)
                G1 = jnp.concatenate([f1b, -f2b], axis=-1)
                G2 = jnp.concatenate([f2b, f1b], axis=-1)
                Ma = jax.lax.dot_general(G1, G2, (((1,), (1,)), ((), ())),
                                         preferred_element_type=f32)
                Tb = jnp.tanh(Ma).astype(bf16)
                Ar_d[b] = jnp.maximum(Tb, 0)
                Acm_d[b] = jnp.minimum(Tb, 0)
                y1 = jnp.concatenate([inpb, ones_col], axis=-1)
                V = jnp.dot(Ar_d[b], y1, preferred_element_type=f32)
                U = jnp.dot(Acm_d[b], y1, preferred_element_type=f32)
                rs_d[b] = 1.0 / (V[:, -1:] + _EPS)
                cs_d[b] = 1.0 / (U[:, -1:] - _EPS)
                h1d1_d[b] = k2(inp, rs_d[b] * V[:, :-1])
                h1d2_d[b] = k2(inp, cs_d[b] * U[:, :-1])

            h2d1_d = {b: k2(inps[b], rs_d[b] * jnp.dot(
                Ar_d[b], h1d1_d[b].astype(bf16), preferred_element_type=f32))
                for b in grp}
            h2d2_d = {b: k2(inps[b], cs_d[b] * jnp.dot(
                Acm_d[b], h1d2_d[b].astype(bf16), preferred_element_type=f32))
                for b in grp}

            cbs = {}
            for b in grp:
                ho = jnp.concatenate(
                    [inps[b], blk(H1f, b), blk(H2f, b), blk(H1b, b),
                     blk(H2b, b), h1d1_d[b], h2d1_d[b], h1d2_d[b],
                     h2d2_d[b]], axis=-1).astype(bf16)
                zr = jax.nn.sigmoid(
                    jnp.dot(ho, wzr, preferred_element_type=f32) + bzr_v)
                z = zr[:, :H]
                r = zr[:, H:]
                zs[b] = z
                cin = jnp.concatenate([xts[b], r * hs[b]], axis=-1)
                cins[b] = cin
                cbs[b] = cin.astype(bf16)

            c1d1_d = {b: k2(cins[b], rs_d[b] * jnp.dot(
                Ar_d[b], cbs[b], preferred_element_type=f32)) for b in grp}
            c1d2_d = {b: k2(cins[b], cs_d[b] * jnp.dot(
                Acm_d[b], cbs[b], preferred_element_type=f32)) for b in grp}
            for b in grp:
                c2d1 = k2(cins[b], rs_d[b] * jnp.dot(
                    Ar_d[b], c1d1_d[b].astype(bf16),
                    preferred_element_type=f32))
                c2d2 = k2(cins[b], cs_d[b] * jnp.dot(
                    Acm_d[b], c1d2_d[b].astype(bf16),
                    preferred_element_type=f32))
                cd1[b] = (c1d1_d[b], c2d1)
                cd2[b] = (c1d2_d[b], c2d2)

        Yc = jnp.concatenate(cins, axis=-1)
        G1f, G2f = static_chain(Afb, sf, Yc)
        G1b, G2b = static_chain(Abb, sb, Yc)

        new_hs = []
        for b in range(B):
            ho = jnp.concatenate(
                [cins[b], blk(G1f, b), blk(G2f, b), blk(G1b, b), blk(G2b, b),
                 cd1[b][0], cd1[b][1], cd2[b][0], cd2[b][1]],
                axis=-1).astype(bf16)
            c = jnp.tanh(jnp.dot(ho, wc, preferred_element_type=f32) + bc_v)
            new_hs.append(zs[b] * hs[b] + (1.0 - zs[b]) * c)
        return tuple(new_hs)

    h0 = tuple(jnp.zeros((N, H), dtype=f32) for _ in range(B))
    hf = jax.lax.fori_loop(0, P, step, h0)
    for b in range(B):
        out_ref[b] = hf[b]


def kernel(x, A_fwd, A_bwd, E1, E2, Wg1, Wg2, Wzr, bzr, Wc, bc):
    B, P, N, C = x.shape
    H = Wc.shape[1]
    EMB = E1.shape[1]
    D = C + H
    feat = Wzr.shape[0]

    xr = x.transpose(1, 2, 0, 3).reshape(P, N, B * C)
    af16 = A_fwd.astype(jnp.bfloat16)
    ab16 = A_bwd.astype(jnp.bfloat16)
    bzr2 = bzr.reshape(1, -1)
    bc2 = bc.reshape(1, -1)

    body = functools.partial(_encoder_body, B=B, P=P, N=N, C=C, H=H)
    out = pl.pallas_call(
        body,
        grid=(1,),
        in_specs=[
            pl.BlockSpec((P, N, B * C), lambda i: (0, 0, 0)),
            pl.BlockSpec((N, N), lambda i: (0, 0)),
            pl.BlockSpec((N, N), lambda i: (0, 0)),
            pl.BlockSpec((N, EMB), lambda i: (0, 0)),
            pl.BlockSpec((N, EMB), lambda i: (0, 0)),
            pl.BlockSpec((D, EMB), lambda i: (0, 0)),
            pl.BlockSpec((D, EMB), lambda i: (0, 0)),
            pl.BlockSpec((feat, 2 * H), lambda i: (0, 0)),
            pl.BlockSpec((1, 2 * H), lambda i: (0, 0)),
            pl.BlockSpec((feat, H), lambda i: (0, 0)),
            pl.BlockSpec((1, H), lambda i: (0, 0)),
        ],
        out_specs=pl.BlockSpec((B, N, H), lambda i: (0, 0, 0)),
        out_shape=jax.ShapeDtypeStruct((B, N, H), x.dtype),
    )(xr, af16, ab16, E1, E2, Wg1, Wg2, Wzr, bzr2, Wc, bc2)
    return out


# speedup vs baseline: 1.6870x; 1.0152x over previous
import functools

import jax
import jax.numpy as jnp
from jax.experimental import pallas as pl
from jax.experimental.pallas import tpu as pltpu

_ALPHA = 0.05
_BETA = 3.0
_K = 2
_EPS = 1e-8


def _encoder_body(x_ref, af_ref, ab_ref, e1_ref, e2_ref, wg1_ref, wg2_ref,
                  wzr_ref, bzr_ref, wc_ref, bc_ref, out_ref, *, B, P, N, C, H):
    f32 = jnp.float32
    bf16 = jnp.bfloat16

    Afb = jnp.maximum(af_ref[...], 0)
    Abb = jnp.maximum(ab_ref[...], 0)
    sf = 1.0 / (jnp.sum(Afb.astype(f32), axis=-1, keepdims=True) + _EPS)
    sb = 1.0 / (jnp.sum(Abb.astype(f32), axis=-1, keepdims=True) + _EPS)

    e1 = e1_ref[...]
    e2 = e2_ref[...]
    wg1 = wg1_ref[...].astype(jnp.bfloat16)
    wg2 = wg2_ref[...].astype(jnp.bfloat16)
    wzr = wzr_ref[...].astype(jnp.bfloat16)
    bzr_v = bzr_ref[...]
    wc = wc_ref[...].astype(jnp.bfloat16)
    bc_v = bc_ref[...]
    ones_col = jnp.ones((N, 1), dtype=bf16)
    D = wg1.shape[0]

    def static_chain(A16, scale, Ym):
        P1 = scale * jnp.dot(A16, Ym.astype(bf16), preferred_element_type=f32)
        H1 = _ALPHA * Ym + (1.0 - _ALPHA) * P1
        P2 = scale * jnp.dot(A16, H1.astype(bf16), preferred_element_type=f32)
        H2 = _ALPHA * Ym + (1.0 - _ALPHA) * P2
        return H1, H2

    def k2(y, p1):
        return _ALPHA * y + (1.0 - _ALPHA) * p1

    def blk(Am, b):
        return Am[:, b * D:(b + 1) * D]

    def step(t, hs):
        xts = [x_ref[t, :, b * C:(b + 1) * C] for b in range(B)]
        inps = [jnp.concatenate([xts[b], hs[b]], axis=-1) for b in range(B)]
        Y0 = jnp.concatenate(inps, axis=-1)
        H1f = H2f = H1b = H2b = None

        zs, cins = [None] * B, [None] * B
        cd1, cd2 = [None] * B, [None] * B
        for g in range(0, B, 2):
            grp = range(g, min(g + 2, B))
            Ar_d, Acm_d, rs_d, cs_d, h1d1_d, h1d2_d = {}, {}, {}, {}, {}, {}
            for b in grp:
                inp = inps[b]
                inpb = inp.astype(bf16)
                f1 = jnp.tanh(jnp.dot(inpb, wg1, preferred_element_type=f32)
                              * e1)
                f2 = jnp.tanh(jnp.dot(inpb, wg2, preferred_element_type=f32)
                              * e2)
                f1b = (_BETA * f1).astype(bf16)
                f2b = f2.astype(bf16)
                G1 = jnp.concatenate([f1b, -f2b], axis=-1)
                G2 = jnp.concatenate([f2b, f1b], axis=-1)
                Ma = jax.lax.dot_general(G1, G2, (((1,), (1,)), ((), ())),
                                         preferred_element_type=f32)
                Tb = jnp.tanh(Ma).astype(bf16)
                Ar_d[b] = jnp.maximum(Tb, 0)
                Acm_d[b] = jnp.minimum(Tb, 0)
                y1 = jnp.concatenate([inpb, ones_col], axis=-1)
                V = jnp.dot(Ar_d[b], y1, preferred_element_type=f32)
                U = jnp.dot(Acm_d[b], y1, preferred_element_type=f32)
                rs_d[b] = 1.0 / (V[:, -1:] + _EPS)
                cs_d[b] = 1.0 / (U[:, -1:] - _EPS)
                h1d1_d[b] = k2(inp, rs_d[b] * V[:, :-1])
                h1d2_d[b] = k2(inp, cs_d[b] * U[:, :-1])

            if g == 0:
                H1f, H2f = static_chain(Afb, sf, Y0)
                H1b, H2b = static_chain(Abb, sb, Y0)

            h2d1_d = {b: k2(inps[b], rs_d[b] * jnp.dot(
                Ar_d[b], h1d1_d[b].astype(bf16), preferred_element_type=f32))
                for b in grp}
            h2d2_d = {b: k2(inps[b], cs_d[b] * jnp.dot(
                Acm_d[b], h1d2_d[b].astype(bf16), preferred_element_type=f32))
                for b in grp}

            cbs = {}
            for b in grp:
                ho = jnp.concatenate(
                    [inps[b], blk(H1f, b), blk(H2f, b), blk(H1b, b),
                     blk(H2b, b), h1d1_d[b], h2d1_d[b], h1d2_d[b],
                     h2d2_d[b]], axis=-1).astype(bf16)
                zr = jax.nn.sigmoid(
                    jnp.dot(ho, wzr, preferred_element_type=f32) + bzr_v)
                z = zr[:, :H]
                r = zr[:, H:]
                zs[b] = z
                cin = jnp.concatenate([xts[b], r * hs[b]], axis=-1)
                cins[b] = cin
                cbs[b] = cin.astype(bf16)

            c1d1_d = {b: k2(cins[b], rs_d[b] * jnp.dot(
                Ar_d[b], cbs[b], preferred_element_type=f32)) for b in grp}
            c1d2_d = {b: k2(cins[b], cs_d[b] * jnp.dot(
                Acm_d[b], cbs[b], preferred_element_type=f32)) for b in grp}
            for b in grp:
                c2d1 = k2(cins[b], rs_d[b] * jnp.dot(
                    Ar_d[b], c1d1_d[b].astype(bf16),
                    preferred_element_type=f32))
                c2d2 = k2(cins[b], cs_d[b] * jnp.dot(
                    Acm_d[b], c1d2_d[b].astype(bf16),
                    preferred_element_type=f32))
                cd1[b] = (c1d1_d[b], c2d1)
                cd2[b] = (c1d2_d[b], c2d2)

        Yc = jnp.concatenate(cins, axis=-1)
        G1f, G2f = static_chain(Afb, sf, Yc)
        G1b, G2b = static_chain(Abb, sb, Yc)

        new_hs = []
        for b in range(B):
            ho = jnp.concatenate(
                [cins[b], blk(G1f, b), blk(G2f, b), blk(G1b, b), blk(G2b, b),
                 cd1[b][0], cd1[b][1], cd2[b][0], cd2[b][1]],
                axis=-1).astype(bf16)
            c = jnp.tanh(jnp.dot(ho, wc, preferred_element_type=f32) + bc_v)
            new_hs.append(zs[b] * hs[b] + (1.0 - zs[b]) * c)
        return tuple(new_hs)

    h0 = tuple(jnp.zeros((N, H), dtype=f32) for _ in range(B))
    hf = jax.lax.fori_loop(0, P, step, h0)
    for b in range(B):
        out_ref[b] = hf[b]


def kernel(x, A_fwd, A_bwd, E1, E2, Wg1, Wg2, Wzr, bzr, Wc, bc):
    B, P, N, C = x.shape
    H = Wc.shape[1]
    EMB = E1.shape[1]
    D = C + H
    feat = Wzr.shape[0]

    xr = x.transpose(1, 2, 0, 3).reshape(P, N, B * C)
    af16 = A_fwd.astype(jnp.bfloat16)
    ab16 = A_bwd.astype(jnp.bfloat16)
    bzr2 = bzr.reshape(1, -1)
    bc2 = bc.reshape(1, -1)

    body = functools.partial(_encoder_body, B=B, P=P, N=N, C=C, H=H)
    out = pl.pallas_call(
        body,
        grid=(1,),
        in_specs=[
            pl.BlockSpec((P, N, B * C), lambda i: (0, 0, 0)),
            pl.BlockSpec((N, N), lambda i: (0, 0)),
            pl.BlockSpec((N, N), lambda i: (0, 0)),
            pl.BlockSpec((N, EMB), lambda i: (0, 0)),
            pl.BlockSpec((N, EMB), lambda i: (0, 0)),
            pl.BlockSpec((D, EMB), lambda i: (0, 0)),
            pl.BlockSpec((D, EMB), lambda i: (0, 0)),
            pl.BlockSpec((feat, 2 * H), lambda i: (0, 0)),
            pl.BlockSpec((1, 2 * H), lambda i: (0, 0)),
            pl.BlockSpec((feat, H), lambda i: (0, 0)),
            pl.BlockSpec((1, H), lambda i: (0, 0)),
        ],
        out_specs=pl.BlockSpec((B, N, H), lambda i: (0, 0, 0)),
        out_shape=jax.ShapeDtypeStruct((B, N, H), x.dtype),
    )(xr, af16, ab16, E1, E2, Wg1, Wg2, Wzr, bzr2, Wc, bc2)
    return out
